# baseline (speedup 1.0000x reference)
import functools

import jax
import jax.numpy as jnp
from jax import lax
from jax.experimental import pallas as pl
from jax.experimental.pallas import tpu as pltpu
from jax.experimental.pallas import tpu_sc as plsc

_EMBED_DIM = 128
_NUM_CORES = 2
_NUM_SUBCORES = 16
_NUM_WORKERS = _NUM_CORES * _NUM_SUBCORES

_CHUNKS = (16, 48, 64, 128, 128, 128, 128)
_RUN = 128


def _make_gather(vocab: int, t: int, n: int, dim: int):
    batch = t * n
    b_per_w = batch // _NUM_WORKERS
    assert sum(_CHUNKS) == b_per_w
    offs = [sum(_CHUNKS[:c]) for c in range(len(_CHUNKS))]
    runs = b_per_w // _RUN
    mesh = plsc.VectorSubcoreMesh(core_axis_name="c", subcore_axis_name="s")

    @functools.partial(
        pl.kernel,
        mesh=mesh,
        out_type=jax.ShapeDtypeStruct((batch, dim), jnp.float32),
        scratch_types=[
            pltpu.VMEM((b_per_w,), jnp.int32),
            pltpu.VMEM((b_per_w, dim), jnp.float32),
            pltpu.VMEM_SHARED((vocab, dim), jnp.float32),
        ]
        + [pltpu.SemaphoreType.DMA] * (len(_CHUNKS) + 3),
        compiler_params=pltpu.CompilerParams(use_tc_tiling_on_sc=True),
    )
    def gather(arrt_hbm, table_hbm, out_hbm, idx_v, rows_v, table_sh, *sems):
        gsems, ssem, stsem, isem = (
            sems[: len(_CHUNKS)],
            sems[len(_CHUNKS)],
            sems[len(_CHUNKS) + 1],
            sems[len(_CHUNKS) + 2],
        )
        sid = lax.axis_index("s")
        wid = sid * _NUM_CORES + lax.axis_index("c")
        base = wid * b_per_w

        @pl.when(sid == 0)
        def _stage():
            pltpu.async_copy(table_hbm, table_sh, stsem)

        idx_copies = []
        for r in range(runs):
            p = base + r * _RUN
            idx_copies.append(
                pltpu.async_copy(
                    arrt_hbm.at[p // n, pl.ds(p % n, _RUN)],
                    idx_v.at[pl.ds(r * _RUN, _RUN)],
                    isem,
                )
            )
        for c in idx_copies:
            c.wait()

        gathers = [
            pltpu.async_copy(
                table_hbm.at[idx_v.at[pl.ds(0, _CHUNKS[0])]],
                rows_v.at[pl.ds(0, _CHUNKS[0])],
                gsems[0],
            )
        ]

        @pl.when(sid == 0)
        def _stage_wait():
            pltpu.make_async_copy(table_hbm, table_sh, stsem).wait()

        plsc.subcore_barrier()

        gathers += [
            pltpu.async_copy(
                table_sh.at[idx_v.at[pl.ds(offs[c], _CHUNKS[c])]],
                rows_v.at[pl.ds(offs[c], _CHUNKS[c])],
                gsems[c],
            )
            for c in range(1, len(_CHUNKS))
        ]
        scatters = []
        for c in range(len(_CHUNKS)):
            gathers[c].wait()
            scatters.append(
                pltpu.async_copy(
                    rows_v.at[pl.ds(offs[c], _CHUNKS[c])],
                    out_hbm.at[pl.ds(base + offs[c], _CHUNKS[c])],
                    ssem,
                )
            )
        for s in scatters:
            s.wait()

    return gather


def kernel(arr, W_s):
    n, t = arr.shape
    idx2d = arr.T.astype(jnp.int32)
    out = _make_gather(W_s.shape[0], t, n, _EMBED_DIM)(idx2d, W_s)
    return out.reshape(t, n, _EMBED_DIM).transpose(1, 0, 2)


# speedup vs baseline: 1.0358x; 1.0358x over previous
import functools

import jax
import jax.numpy as jnp
from jax import lax
from jax.experimental import pallas as pl
from jax.experimental.pallas import tpu as pltpu
from jax.experimental.pallas import tpu_sc as plsc

_EMBED_DIM = 128
_NUM_CORES = 2
_NUM_SUBCORES = 16
_NUM_WORKERS = _NUM_CORES * _NUM_SUBCORES

_CHUNKS = (80, 80, 80, 80, 80, 80, 80, 80)
_RUN = 128


def _make_gather(vocab: int, t: int, n: int, dim: int):
    batch = t * n
    b_per_w = batch // _NUM_WORKERS
    assert sum(_CHUNKS) == b_per_w
    offs = [sum(_CHUNKS[:c]) for c in range(len(_CHUNKS))]
    runs = b_per_w // _RUN
    mesh = plsc.VectorSubcoreMesh(core_axis_name="c", subcore_axis_name="s")

    @functools.partial(
        pl.kernel,
        mesh=mesh,
        out_type=jax.ShapeDtypeStruct((batch, dim), jnp.float32),
        scratch_types=[
            pltpu.VMEM((b_per_w,), jnp.int32),
            pltpu.VMEM((b_per_w, dim), jnp.float32),
            pltpu.VMEM_SHARED((vocab, dim), jnp.float32),
        ]
        + [pltpu.SemaphoreType.DMA] * (len(_CHUNKS) + 3),
        compiler_params=pltpu.CompilerParams(use_tc_tiling_on_sc=True),
    )
    def gather(arrt_hbm, table_hbm, out_hbm, idx_v, rows_v, table_sh, *sems):
        gsems, ssem, stsem, isem = (
            sems[: len(_CHUNKS)],
            sems[len(_CHUNKS)],
            sems[len(_CHUNKS) + 1],
            sems[len(_CHUNKS) + 2],
        )
        sid = lax.axis_index("s")
        wid = sid * _NUM_CORES + lax.axis_index("c")
        base = wid * b_per_w

        @pl.when(sid == 0)
        def _stage():
            pltpu.async_copy(table_hbm, table_sh, stsem)

        idx_copies = []
        for r in range(runs):
            p = base + r * _RUN
            idx_copies.append(
                pltpu.async_copy(
                    arrt_hbm.at[p // n, pl.ds(p % n, _RUN)],
                    idx_v.at[pl.ds(r * _RUN, _RUN)],
                    isem,
                )
            )
        for c in idx_copies:
            c.wait()

        gathers = [
            pltpu.async_copy(
                table_hbm.at[idx_v.at[pl.ds(0, _CHUNKS[0])]],
                rows_v.at[pl.ds(0, _CHUNKS[0])],
                gsems[0],
            )
        ]

        @pl.when(sid == 0)
        def _stage_wait():
            pltpu.make_async_copy(table_hbm, table_sh, stsem).wait()

        plsc.subcore_barrier()

        gathers += [
            pltpu.async_copy(
                table_sh.at[idx_v.at[pl.ds(offs[c], _CHUNKS[c])]],
                rows_v.at[pl.ds(offs[c], _CHUNKS[c])],
                gsems[c],
            )
            for c in range(1, len(_CHUNKS))
        ]
        scatters = []
        for c in range(len(_CHUNKS)):
            gathers[c].wait()
            scatters.append(
                pltpu.async_copy(
                    rows_v.at[pl.ds(offs[c], _CHUNKS[c])],
                    out_hbm.at[pl.ds(base + offs[c], _CHUNKS[c])],
                    ssem,
                )
            )
        for s in scatters:
            s.wait()

    return gather


def kernel(arr, W_s):
    n, t = arr.shape
    idx2d = arr.T.astype(jnp.int32)
    out = _make_gather(W_s.shape[0], t, n, _EMBED_DIM)(idx2d, W_s)
    return out.reshape(t, n, _EMBED_DIM).transpose(1, 0, 2)
